# baseline (speedup 1.0000x reference)
import functools

import jax
import jax.numpy as jnp
from jax import lax
from jax.experimental import pallas as pl
from jax.experimental.pallas import tpu as pltpu
from jax.experimental.pallas import tpu_sc as plsc

LANES = 16
CHUNK = 4

_GATHER_DNUMS = lax.GatherDimensionNumbers(
    offset_dims=(), collapsed_slice_dims=(0,), start_index_map=(0,)
)


def _lane_broadcast(vec, lane):
    idx = jnp.full((LANES,), lane, jnp.int32).reshape(LANES, 1)
    return lax.gather(
        vec,
        idx,
        dimension_numbers=_GATHER_DNUMS,
        slice_sizes=(1,),
        mode=lax.GatherScatterMode.PROMISE_IN_BOUNDS,
    )


def _make_sc_kernel(seq_len, batch, dim, num_peaks):
    info = plsc.get_sparse_core_info()
    n_workers = info.num_cores * info.num_subcores
    rows_per_worker = seq_len // n_workers
    n_chunks = rows_per_worker // CHUNK
    vecs = dim // LANES
    groups = batch // LANES
    mesh = plsc.VectorSubcoreMesh(core_axis_name="c", subcore_axis_name="s")

    @functools.partial(
        pl.kernel,
        mesh=mesh,
        out_type=jax.ShapeDtypeStruct((seq_len, batch, dim), jnp.float32),
        scratch_types=[
            pltpu.VMEM((CHUNK, batch, dim), jnp.float32),
            pltpu.VMEM((CHUNK, dim), jnp.float32),
            pltpu.VMEM((CHUNK, dim), jnp.float32),
            pltpu.VMEM((num_peaks, batch), jnp.int32),
        ],
    )
    def sc_kernel(x_hbm, pos_hbm, pe_hbm, tab_hbm, out_hbm, xbuf, pebuf, tabbuf, posv):
        wid = lax.axis_index("s") * info.num_cores + lax.axis_index("c")
        row0 = wid * rows_per_worker
        pltpu.sync_copy(pos_hbm, posv)

        def chunk_body(ci, _):
            s_base = row0 + ci * CHUNK
            pltpu.sync_copy(x_hbm.at[pl.ds(s_base, CHUNK)], xbuf)
            pltpu.sync_copy(pe_hbm.at[pl.ds(s_base, CHUNK)], pebuf)
            pltpu.sync_copy(tab_hbm.at[pl.ds(s_base, CHUNK)], tabbuf)

            def row_body(si, _):
                s = s_base + si
                sv = jnp.full((LANES,), s, jnp.int32)
                pev = [pebuf[si, pl.ds(j * LANES, LANES)] for j in range(vecs)]
                tabv = [tabbuf[si, pl.ds(j * LANES, LANES)] for j in range(vecs)]

                def group_body(g, _):
                    cg = jnp.zeros((LANES,), jnp.float32)
                    for k in range(num_peaks):
                        pk = posv[k, pl.ds(g * LANES, LANES)]
                        cg = cg + jnp.where(pk == sv, 1.0, 0.0)

                    def lane_body(i, _):
                        b = g * LANES + i
                        cb = _lane_broadcast(cg, i)
                        for j in range(vecs):
                            sl = pl.ds(j * LANES, LANES)
                            xbuf[si, b, sl] = (
                                xbuf[si, b, sl] + pev[j] + cb * tabv[j]
                            )
                        return 0

                    lax.fori_loop(0, LANES, lane_body, 0)
                    return 0

                lax.fori_loop(0, groups, group_body, 0)
                return 0

            lax.fori_loop(0, CHUNK, row_body, 0)
            pltpu.sync_copy(xbuf, out_hbm.at[pl.ds(s_base, CHUNK)])
            return 0

        lax.fori_loop(0, n_chunks, chunk_body, 0)

    return sc_kernel


def kernel(x, peak_positions, pe, peak_table):
    seq_len, batch, dim = x.shape
    num_peaks = peak_positions.shape[1]
    sck = _make_sc_kernel(seq_len, batch, dim, num_peaks)
    return sck(x, peak_positions.T, pe[:seq_len], peak_table[:seq_len])


# speedup vs baseline: 7.0034x; 7.0034x over previous
import jax
import jax.numpy as jnp
from jax.experimental import pallas as pl
from jax.experimental.pallas import tpu as pltpu

SEQ_BLOCK = 128


def _body(pos_ref, x_ref, pet_ref, tabt_ref, o_ref):
    i = pl.program_id(0)
    s_blk, dim, batch = x_ref.shape
    s_ids = i * s_blk + jax.lax.broadcasted_iota(jnp.int32, (s_blk, 1, 1), 0)
    pos = pos_ref[...]
    c = jnp.zeros((s_blk, 1, batch), jnp.float32)
    for k in range(pos_ref.shape[0]):
        c = c + (s_ids == pos[k][None, None, :]).astype(jnp.float32)
    off = pl.multiple_of(i * s_blk, 128)
    pe = pet_ref[:, pl.ds(off, s_blk)].T
    tab = tabt_ref[:, pl.ds(off, s_blk)].T
    o_ref[...] = x_ref[...] + pe[:, :, None] + c * tab[:, :, None]


def kernel(x, peak_positions, pe, peak_table):
    seq_len, batch, dim = x.shape
    num_peaks = peak_positions.shape[1]
    xt = x.transpose(0, 2, 1)
    pos_t = peak_positions.T
    pet = pe[:seq_len].T
    tabt = peak_table[:seq_len].T
    grid = (seq_len // SEQ_BLOCK,)
    out_t = pl.pallas_call(
        _body,
        grid=grid,
        in_specs=[
            pl.BlockSpec((num_peaks, batch), lambda i: (0, 0)),
            pl.BlockSpec((SEQ_BLOCK, dim, batch), lambda i: (i, 0, 0)),
            pl.BlockSpec((dim, seq_len), lambda i: (0, 0)),
            pl.BlockSpec((dim, seq_len), lambda i: (0, 0)),
        ],
        out_specs=pl.BlockSpec((SEQ_BLOCK, dim, batch), lambda i: (i, 0, 0)),
        out_shape=jax.ShapeDtypeStruct(xt.shape, x.dtype),
        compiler_params=pltpu.CompilerParams(
            dimension_semantics=("parallel",),
        ),
    )(pos_t, xt, pet, tabt)
    return out_t.transpose(0, 2, 1)


# speedup vs baseline: 7.4066x; 1.0576x over previous
import jax
import jax.numpy as jnp
from jax.experimental import pallas as pl
from jax.experimental.pallas import tpu as pltpu

SEQ_BLOCK = 256


def _body(pos_ref, x_ref, pet_ref, tabt_ref, o_ref):
    i = pl.program_id(0)
    s_blk, dim, batch = x_ref.shape
    s_ids = i * s_blk + jax.lax.broadcasted_iota(jnp.int32, (s_blk, 1, 1), 0)
    pos = pos_ref[...]
    c = jnp.zeros((s_blk, 1, batch), jnp.float32)
    for k in range(pos_ref.shape[0]):
        c = c + (s_ids == pos[k][None, None, :]).astype(jnp.float32)
    off = pl.multiple_of(i * s_blk, 128)
    pe = pet_ref[:, pl.ds(off, s_blk)].T
    tab = tabt_ref[:, pl.ds(off, s_blk)].T
    o_ref[...] = x_ref[...] + pe[:, :, None] + c * tab[:, :, None]


def kernel(x, peak_positions, pe, peak_table):
    seq_len, batch, dim = x.shape
    num_peaks = peak_positions.shape[1]
    xt = x.transpose(0, 2, 1)
    pos_t = peak_positions.T
    pet = pe[:seq_len].T
    tabt = peak_table[:seq_len].T
    grid = (seq_len // SEQ_BLOCK,)
    out_t = pl.pallas_call(
        _body,
        grid=grid,
        in_specs=[
            pl.BlockSpec((num_peaks, batch), lambda i: (0, 0)),
            pl.BlockSpec((SEQ_BLOCK, dim, batch), lambda i: (i, 0, 0)),
            pl.BlockSpec((dim, seq_len), lambda i: (0, 0)),
            pl.BlockSpec((dim, seq_len), lambda i: (0, 0)),
        ],
        out_specs=pl.BlockSpec((SEQ_BLOCK, dim, batch), lambda i: (i, 0, 0)),
        out_shape=jax.ShapeDtypeStruct(xt.shape, x.dtype),
        compiler_params=pltpu.CompilerParams(
            dimension_semantics=("parallel",),
        ),
    )(pos_t, xt, pet, tabt)
    return out_t.transpose(0, 2, 1)
